# baseline (speedup 1.0000x reference)
import functools

import jax
import jax.numpy as jnp
from jax import lax
from jax.experimental import pallas as pl
from jax.experimental.pallas import tpu as pltpu
from jax.experimental.pallas import tpu_sc as plsc

N = 10000
E = 320000
F = 128
NC = 2
NS = 16
NW = NC * NS
NP = 10240
RPT = NP // NS
EC = 128
NCHUNK = 80
EPT = EC * NCHUNK
EPAD = NW * EPT

_mesh = plsc.VectorSubcoreMesh(
    core_axis_name="c", subcore_axis_name="s", num_cores=NC, num_subcores=NS)


@functools.partial(
    pl.kernel,
    out_type=(
        jax.ShapeDtypeStruct((NW * NP,), jnp.float32),
        jax.ShapeDtypeStruct((NW * NP,), jnp.float32),
    ),
    mesh=_mesh,
    compiler_params=pltpu.CompilerParams(needs_layout_passes=False),
    scratch_types=[
        pltpu.VMEM((EPT,), jnp.int32),
        pltpu.VMEM((EPT,), jnp.int32),
        pltpu.VMEM((NP,), jnp.float32),
        pltpu.VMEM((NP,), jnp.float32),
    ],
)
def _deg_kernel(src_hbm, dst_hbm, zeros_hbm, outO_hbm, outI_hbm,
                sidx, didx, accO, accI):
    cid = lax.axis_index("c")
    sid = lax.axis_index("s")
    wid = cid * NS + sid

    pltpu.sync_copy(zeros_hbm, accO)
    pltpu.sync_copy(zeros_hbm, accI)
    pltpu.sync_copy(src_hbm.at[pl.ds(wid * EPT, EPT)], sidx)
    pltpu.sync_copy(dst_hbm.at[pl.ds(wid * EPT, EPT)], didx)

    one16 = jnp.ones((16,), jnp.float32)

    def body(i, carry):
        iv = sidx[pl.ds(i * 16, 16)]
        plsc.addupdate_scatter(accO, [iv], one16)
        jv = didx[pl.ds(i * 16, 16)]
        plsc.addupdate_scatter(accI, [jv], one16)
        return carry

    lax.fori_loop(0, EPT // 16, body, 0)

    pltpu.sync_copy(accO, outO_hbm.at[pl.ds(wid * NP, NP)])
    pltpu.sync_copy(accI, outI_hbm.at[pl.ds(wid * NP, NP)])


@functools.partial(
    pl.kernel,
    out_type=jax.ShapeDtypeStruct((NC, NP, F), jnp.float32),
    mesh=_mesh,
    scratch_types=[
        pltpu.VMEM((EPT,), jnp.int32),
        pltpu.VMEM((NCHUNK, EC), jnp.int32),
        pltpu.VMEM((EC, F), jnp.float32),
        pltpu.VMEM_SHARED((NP, F), jnp.float32),
        pltpu.SemaphoreType.DMA,
    ],
)
def _agg_kernel(src_hbm, dst_hbm, table_hbm, zeros_hbm, out_hbm,
                sidx, didx, rows0, acc, sem0):
    cid = lax.axis_index("c")
    sid = lax.axis_index("s")
    wid = cid * NS + sid

    row0 = sid * RPT
    pltpu.sync_copy(zeros_hbm.at[pl.ds(row0, RPT)], acc.at[pl.ds(row0, RPT)])
    pltpu.sync_copy(src_hbm.at[pl.ds(wid * EPT, EPT)], sidx)
    pltpu.sync_copy(dst_hbm.at[wid], didx)
    plsc.subcore_barrier()

    def start_gather(j, buf, sem):
        pltpu.async_copy(table_hbm.at[sidx.at[pl.ds(j * EC, EC)]], buf, sem)

    def wait_gather(buf, sem):
        pltpu.make_async_copy(
            table_hbm.at[sidx.at[pl.ds(0, EC)]], buf, sem).wait()

    def scatter(j, buf):
        pltpu.sync_copy(buf, acc.at[didx.at[j]], add=True)

    def body(j, carry):
        start_gather(j, rows0, sem0)
        wait_gather(rows0, sem0)
        scatter(j, rows0)
        return carry

    lax.fori_loop(0, NCHUNK, body, 0)
    plsc.subcore_barrier()

    pltpu.sync_copy(acc.at[pl.ds(row0, RPT)], out_hbm.at[cid, pl.ds(row0, RPT)])


_RB = 2048
_GRID = NP // _RB


def _deg_to_scale(dp_ref):
    d = jnp.sum(dp_ref[...], axis=0)
    return lax.rsqrt(jnp.maximum(d, 1.0))


def _scale_x_body(x_ref, dOp_ref, xa_ref):
    a = _deg_to_scale(dOp_ref)
    xa_ref[...] = x_ref[...] * a[:, None]


def _mid_body(aggp_ref, dOp_ref, dIp_ref, West_ref, best_ref, W1_ref, b1_ref,
              s_ref, h1a_ref):
    b = _deg_to_scale(dIp_ref)
    agg = (aggp_ref[0] + aggp_ref[1]) * b[:, None]
    s_ref[...] = jnp.dot(agg, West_ref[...],
                         preferred_element_type=jnp.float32) + best_ref[...]
    a = _deg_to_scale(dOp_ref)
    h1 = jnp.maximum(
        jnp.dot(agg, W1_ref[...], preferred_element_type=jnp.float32)
        + b1_ref[...], 0.0)
    h1a_ref[...] = h1 * a[:, None]


def _final_body(aggp_ref, dIp_ref, W2_ref, b2_ref, Wc_ref, bc_ref,
                z_ref, y_ref):
    b = _deg_to_scale(dIp_ref)
    agg = (aggp_ref[0] + aggp_ref[1]) * b[:, None]
    z = jnp.dot(agg, W2_ref[...],
                preferred_element_type=jnp.float32) + b2_ref[...]
    z_ref[...] = z
    y_ref[...] = jnp.dot(z, Wc_ref[...],
                         preferred_element_type=jnp.float32) + bc_ref[...]


def _row_spec(width):
    return pl.BlockSpec((_RB, width), lambda i: (i, 0))


def _degp_spec():
    return pl.BlockSpec((NW, _RB), lambda i: (0, i))


def _aggp_spec():
    return pl.BlockSpec((NC, _RB, F), lambda i: (0, i, 0))


def _full(shape):
    return pl.BlockSpec(shape, lambda i: tuple(0 for _ in shape))


def kernel(g, x, W_est, b_est, W1, b1, W2, b2, Wc, bc):
    src = jnp.asarray(g[0], jnp.int32)
    dst = jnp.asarray(g[1], jnp.int32)
    pad = jnp.full((EPAD - E,), NP - 1, jnp.int32)
    srcp = jnp.concatenate([src, pad])
    dstp = jnp.concatenate([dst, pad])
    dst3 = dstp.reshape(NW, NCHUNK, EC)

    zeros_deg = jnp.zeros((NP,), jnp.float32)
    zeros_f = jnp.zeros((NP, F), jnp.float32)

    degO_p, degI_p = _deg_kernel(srcp, dstp, zeros_deg)
    degO_p = degO_p.reshape(NW, NP)
    degI_p = degI_p.reshape(NW, NP)

    xa = pl.pallas_call(
        _scale_x_body,
        grid=(_GRID,),
        in_specs=[_row_spec(F), _degp_spec()],
        out_specs=_row_spec(F),
        out_shape=jax.ShapeDtypeStruct((NP, F), jnp.float32),
    )(x, degO_p)

    agg0_p = _agg_kernel(srcp, dst3, xa, zeros_f)

    s, h1a = pl.pallas_call(
        _mid_body,
        grid=(_GRID,),
        in_specs=[
            _aggp_spec(), _degp_spec(), _degp_spec(),
            _full((F, 1)), _full((1, 1)), _full((F, F)), _full((1, F)),
        ],
        out_specs=[_row_spec(1), _row_spec(F)],
        out_shape=[
            jax.ShapeDtypeStruct((N, 1), jnp.float32),
            jax.ShapeDtypeStruct((NP, F), jnp.float32),
        ],
    )(agg0_p, degO_p, degI_p, W_est, b_est.reshape(1, 1), W1, b1.reshape(1, F))

    agg1_p = _agg_kernel(srcp, dst3, h1a, zeros_f)

    z, y = pl.pallas_call(
        _final_body,
        grid=(_GRID,),
        in_specs=[
            _aggp_spec(), _degp_spec(),
            _full((F, F)), _full((1, F)), _full((F, 1)), _full((1, 1)),
        ],
        out_specs=[_row_spec(F), _row_spec(1)],
        out_shape=[
            jax.ShapeDtypeStruct((N, F), jnp.float32),
            jax.ShapeDtypeStruct((N, 1), jnp.float32),
        ],
    )(agg1_p, degI_p, W2, b2.reshape(1, F), Wc, bc.reshape(1, 1))

    return (s, z, y)


# speedup vs baseline: 2.5438x; 2.5438x over previous
import functools

import jax
import jax.numpy as jnp
from jax import lax
from jax.experimental import pallas as pl
from jax.experimental.pallas import tpu as pltpu
from jax.experimental.pallas import tpu_sc as plsc

N = 10000
E = 320000
F = 128
NC = 2
NS = 16
NW = NC * NS
NP = 10240
RPT = NP // NS
EC = 80
NCHUNK = 125
EPT = EC * NCHUNK

_mesh = plsc.VectorSubcoreMesh(
    core_axis_name="c", subcore_axis_name="s", num_cores=NC, num_subcores=NS)


@functools.partial(
    pl.kernel,
    out_type=(
        jax.ShapeDtypeStruct((NW * NP,), jnp.float32),
        jax.ShapeDtypeStruct((NW * NP,), jnp.float32),
    ),
    mesh=_mesh,
    compiler_params=pltpu.CompilerParams(needs_layout_passes=False),
    scratch_types=[
        pltpu.VMEM((EPT,), jnp.int32),
        pltpu.VMEM((EPT,), jnp.int32),
        pltpu.VMEM((NP,), jnp.float32),
        pltpu.VMEM((NP,), jnp.float32),
    ],
)
def _deg_kernel(src_hbm, dst_hbm, zeros_hbm, outO_hbm, outI_hbm,
                sidx, didx, accO, accI):
    cid = lax.axis_index("c")
    sid = lax.axis_index("s")
    wid = cid * NS + sid

    pltpu.sync_copy(zeros_hbm, accO)
    pltpu.sync_copy(zeros_hbm, accI)
    pltpu.sync_copy(src_hbm.at[pl.ds(wid * EPT, EPT)], sidx)
    pltpu.sync_copy(dst_hbm.at[pl.ds(wid * EPT, EPT)], didx)

    one16 = jnp.ones((16,), jnp.float32)

    def body(i, carry):
        iv = sidx[pl.ds(i * 16, 16)]
        plsc.addupdate_scatter(accO, [iv], one16)
        jv = didx[pl.ds(i * 16, 16)]
        plsc.addupdate_scatter(accI, [jv], one16)
        return carry

    lax.fori_loop(0, EPT // 16, body, 0)

    pltpu.sync_copy(accO, outO_hbm.at[pl.ds(wid * NP, NP)])
    pltpu.sync_copy(accI, outI_hbm.at[pl.ds(wid * NP, NP)])


@functools.partial(
    pl.kernel,
    out_type=jax.ShapeDtypeStruct((NC, NP, F), jnp.float32),
    mesh=_mesh,
    scratch_types=[
        pltpu.VMEM((EC,), jnp.int32),
        pltpu.VMEM((EC,), jnp.int32),
        pltpu.VMEM((EC,), jnp.int32),
        pltpu.VMEM((EC,), jnp.int32),
        pltpu.VMEM((EC, F), jnp.float32),
        pltpu.VMEM((EC, F), jnp.float32),
        pltpu.VMEM_SHARED((NP, F), jnp.float32),
        pltpu.SemaphoreType.DMA,
        pltpu.SemaphoreType.DMA,
    ],
)
def _agg_kernel(src_hbm, dst_hbm, table_hbm, zeros_hbm, out_hbm,
                sidxA, didxA, sidxB, didxB, rowsA, rowsB, acc, semA, semB):
    cid = lax.axis_index("c")
    sid = lax.axis_index("s")
    wid = cid * NS + sid

    row0 = sid * RPT
    pltpu.sync_copy(zeros_hbm.at[pl.ds(row0, RPT)], acc.at[pl.ds(row0, RPT)])
    plsc.subcore_barrier()

    ebase = wid * EPT

    def load_idx(j, sbuf, dbuf):
        off = ebase + j * EC
        pltpu.sync_copy(src_hbm.at[pl.ds(off, EC)], sbuf)
        pltpu.sync_copy(dst_hbm.at[pl.ds(off, EC)], dbuf)

    def start_gather(sbuf, buf, sem):
        pltpu.async_copy(table_hbm.at[sbuf], buf, sem)

    def wait_gather(buf, sem):
        pltpu.make_async_copy(table_hbm.at[sidxA], buf, sem).wait()

    def scatter(dbuf, buf):
        pltpu.sync_copy(buf, acc.at[dbuf], add=True)

    load_idx(0, sidxA, didxA)
    start_gather(sidxA, rowsA, semA)

    def body(j2, carry):
        j = 2 * j2
        load_idx(j + 1, sidxB, didxB)
        start_gather(sidxB, rowsB, semB)
        wait_gather(rowsA, semA)
        scatter(didxA, rowsA)
        load_idx(j + 2, sidxA, didxA)
        start_gather(sidxA, rowsA, semA)
        wait_gather(rowsB, semB)
        scatter(didxB, rowsB)
        return carry

    lax.fori_loop(0, (NCHUNK - 1) // 2, body, 0)
    wait_gather(rowsA, semA)
    scatter(didxA, rowsA)
    plsc.subcore_barrier()

    pltpu.sync_copy(acc.at[pl.ds(row0, RPT)], out_hbm.at[cid, pl.ds(row0, RPT)])


_RB = 2048
_GRID = NP // _RB


def _deg_to_scale(dp_ref):
    d = jnp.sum(dp_ref[...], axis=0)
    return lax.rsqrt(jnp.maximum(d, 1.0))


def _scale_x_body(x_ref, dOp_ref, xa_ref):
    a = _deg_to_scale(dOp_ref)
    xa_ref[...] = x_ref[...] * a[:, None]


def _mid_body(aggp_ref, dOp_ref, dIp_ref, West_ref, best_ref, W1_ref, b1_ref,
              s_ref, h1a_ref):
    b = _deg_to_scale(dIp_ref)
    agg = (aggp_ref[0] + aggp_ref[1]) * b[:, None]
    s_ref[...] = jnp.dot(agg, West_ref[...],
                         preferred_element_type=jnp.float32) + best_ref[...]
    a = _deg_to_scale(dOp_ref)
    h1 = jnp.maximum(
        jnp.dot(agg, W1_ref[...], preferred_element_type=jnp.float32)
        + b1_ref[...], 0.0)
    h1a_ref[...] = h1 * a[:, None]


def _final_body(aggp_ref, dIp_ref, W2_ref, b2_ref, Wc_ref, bc_ref,
                z_ref, y_ref):
    b = _deg_to_scale(dIp_ref)
    agg = (aggp_ref[0] + aggp_ref[1]) * b[:, None]
    z = jnp.dot(agg, W2_ref[...],
                preferred_element_type=jnp.float32) + b2_ref[...]
    z_ref[...] = z
    y_ref[...] = jnp.dot(z, Wc_ref[...],
                         preferred_element_type=jnp.float32) + bc_ref[...]


def _row_spec(width):
    return pl.BlockSpec((_RB, width), lambda i: (i, 0))


def _degp_spec():
    return pl.BlockSpec((NW, _RB), lambda i: (0, i))


def _aggp_spec():
    return pl.BlockSpec((NC, _RB, F), lambda i: (0, i, 0))


def _full(shape):
    return pl.BlockSpec(shape, lambda i: tuple(0 for _ in shape))


def kernel(g, x, W_est, b_est, W1, b1, W2, b2, Wc, bc):
    src = jnp.asarray(g[0], jnp.int32)
    dst = jnp.asarray(g[1], jnp.int32)
    zeros_deg = jnp.zeros((NP,), jnp.float32)
    zeros_f = jnp.zeros((NP, F), jnp.float32)

    degO_p, degI_p = _deg_kernel(src, dst, zeros_deg)
    degO_p = degO_p.reshape(NW, NP)
    degI_p = degI_p.reshape(NW, NP)

    xa = pl.pallas_call(
        _scale_x_body,
        grid=(_GRID,),
        in_specs=[_row_spec(F), _degp_spec()],
        out_specs=_row_spec(F),
        out_shape=jax.ShapeDtypeStruct((NP, F), jnp.float32),
    )(x, degO_p)

    agg0_p = _agg_kernel(src, dst, xa, zeros_f)

    s, h1a = pl.pallas_call(
        _mid_body,
        grid=(_GRID,),
        in_specs=[
            _aggp_spec(), _degp_spec(), _degp_spec(),
            _full((F, 1)), _full((1, 1)), _full((F, F)), _full((1, F)),
        ],
        out_specs=[_row_spec(1), _row_spec(F)],
        out_shape=[
            jax.ShapeDtypeStruct((N, 1), jnp.float32),
            jax.ShapeDtypeStruct((NP, F), jnp.float32),
        ],
    )(agg0_p, degO_p, degI_p, W_est, b_est.reshape(1, 1), W1, b1.reshape(1, F))

    agg1_p = _agg_kernel(src, dst, h1a, zeros_f)

    z, y = pl.pallas_call(
        _final_body,
        grid=(_GRID,),
        in_specs=[
            _aggp_spec(), _degp_spec(),
            _full((F, F)), _full((1, F)), _full((F, 1)), _full((1, 1)),
        ],
        out_specs=[_row_spec(F), _row_spec(1)],
        out_shape=[
            jax.ShapeDtypeStruct((N, F), jnp.float32),
            jax.ShapeDtypeStruct((N, 1), jnp.float32),
        ],
    )(agg1_p, degI_p, W2, b2.reshape(1, F), Wc, bc.reshape(1, 1))

    return (s, z, y)


# speedup vs baseline: 3.5337x; 1.3891x over previous
import functools

import jax
import jax.numpy as jnp
from jax import lax
from jax.experimental import pallas as pl
from jax.experimental.pallas import tpu as pltpu
from jax.experimental.pallas import tpu_sc as plsc

N = 10000
E = 320000
F = 128
NC = 2
NS = 16
NW = NC * NS
NP = 10240
RPT = NP // NS
EC = 80
NCHUNK = 125
EPT = EC * NCHUNK

_mesh = plsc.VectorSubcoreMesh(
    core_axis_name="c", subcore_axis_name="s", num_cores=NC, num_subcores=NS)


@functools.partial(
    pl.kernel,
    out_type=(
        jax.ShapeDtypeStruct((NW * NP,), jnp.float32),
        jax.ShapeDtypeStruct((NW * NP,), jnp.float32),
    ),
    mesh=_mesh,
    compiler_params=pltpu.CompilerParams(needs_layout_passes=False),
    scratch_types=[
        pltpu.VMEM((EPT,), jnp.int32),
        pltpu.VMEM((EPT,), jnp.int32),
        pltpu.VMEM((NP,), jnp.float32),
        pltpu.VMEM((NP,), jnp.float32),
    ],
)
def _deg_kernel(src_hbm, dst_hbm, zeros_hbm, outO_hbm, outI_hbm,
                sidx, didx, accO, accI):
    cid = lax.axis_index("c")
    sid = lax.axis_index("s")
    wid = cid * NS + sid

    pltpu.sync_copy(zeros_hbm, accO)
    pltpu.sync_copy(zeros_hbm, accI)
    pltpu.sync_copy(src_hbm.at[pl.ds(wid * EPT, EPT)], sidx)
    pltpu.sync_copy(dst_hbm.at[pl.ds(wid * EPT, EPT)], didx)

    one16 = jnp.ones((16,), jnp.float32)

    def body(i, carry):
        iv = sidx[pl.ds(i * 16, 16)]
        plsc.addupdate_scatter(accO, [iv], one16)
        jv = didx[pl.ds(i * 16, 16)]
        plsc.addupdate_scatter(accI, [jv], one16)
        return carry

    lax.fori_loop(0, EPT // 16, body, 0)

    pltpu.sync_copy(accO, outO_hbm.at[pl.ds(wid * NP, NP)])
    pltpu.sync_copy(accI, outI_hbm.at[pl.ds(wid * NP, NP)])


@functools.partial(
    pl.kernel,
    out_type=jax.ShapeDtypeStruct((NC, NP, F), jnp.float32),
    mesh=_mesh,
    scratch_types=[
        pltpu.VMEM((EPT,), jnp.int32),
        pltpu.VMEM((EPT,), jnp.int32),
        pltpu.VMEM((EC, F), jnp.float32),
        pltpu.VMEM((EC, F), jnp.float32),
        pltpu.VMEM_SHARED((NP, F), jnp.float32),
        pltpu.SemaphoreType.DMA,
        pltpu.SemaphoreType.DMA,
    ],
)
def _agg_kernel(src_hbm, dst_hbm, table_hbm, zeros_hbm, out_hbm,
                sidx, didx, rowsA, rowsB, acc, semA, semB):
    cid = lax.axis_index("c")
    sid = lax.axis_index("s")
    wid = cid * NS + sid

    row0 = sid * RPT
    pltpu.sync_copy(zeros_hbm.at[pl.ds(row0, RPT)], acc.at[pl.ds(row0, RPT)])
    pltpu.sync_copy(src_hbm.at[pl.ds(wid * EPT, EPT)], sidx)
    pltpu.sync_copy(dst_hbm.at[pl.ds(wid * EPT, EPT)], didx)
    plsc.subcore_barrier()

    def start_gather(j, buf, sem):
        pltpu.async_copy(table_hbm.at[sidx.at[pl.ds(j * EC, EC)]], buf, sem)

    def wait_gather(j, buf, sem):
        pltpu.make_async_copy(
            table_hbm.at[sidx.at[pl.ds(j * EC, EC)]], buf, sem).wait()

    def scatter(j, buf):
        pltpu.sync_copy(buf, acc.at[didx.at[pl.ds(j * EC, EC)]], add=True)

    start_gather(0, rowsA, semA)

    def body(j2, carry):
        j = 2 * j2
        start_gather(j + 1, rowsB, semB)
        wait_gather(j, rowsA, semA)
        scatter(j, rowsA)
        start_gather(j + 2, rowsA, semA)
        wait_gather(j + 1, rowsB, semB)
        scatter(j + 1, rowsB)
        return carry

    lax.fori_loop(0, (NCHUNK - 1) // 2, body, 0)
    wait_gather(NCHUNK - 1, rowsA, semA)
    scatter(NCHUNK - 1, rowsA)
    plsc.subcore_barrier()

    pltpu.sync_copy(acc.at[pl.ds(row0, RPT)], out_hbm.at[cid, pl.ds(row0, RPT)])


_RB = 2048
_GRID = NP // _RB


def _deg_to_scale(dp_ref):
    d = jnp.sum(dp_ref[...], axis=0)
    return lax.rsqrt(jnp.maximum(d, 1.0))


def _scale_x_body(x_ref, dOp_ref, xa_ref):
    a = _deg_to_scale(dOp_ref)
    xa_ref[...] = x_ref[...] * a[:, None]


def _mid_body(aggp_ref, dOp_ref, dIp_ref, West_ref, best_ref, W1_ref, b1_ref,
              s_ref, h1a_ref):
    b = _deg_to_scale(dIp_ref)
    agg = (aggp_ref[0] + aggp_ref[1]) * b[:, None]
    s_ref[...] = jnp.dot(agg, West_ref[...],
                         preferred_element_type=jnp.float32) + best_ref[...]
    a = _deg_to_scale(dOp_ref)
    h1 = jnp.maximum(
        jnp.dot(agg, W1_ref[...], preferred_element_type=jnp.float32)
        + b1_ref[...], 0.0)
    h1a_ref[...] = h1 * a[:, None]


def _final_body(aggp_ref, dIp_ref, W2_ref, b2_ref, Wc_ref, bc_ref,
                z_ref, y_ref):
    b = _deg_to_scale(dIp_ref)
    agg = (aggp_ref[0] + aggp_ref[1]) * b[:, None]
    z = jnp.dot(agg, W2_ref[...],
                preferred_element_type=jnp.float32) + b2_ref[...]
    z_ref[...] = z
    y_ref[...] = jnp.dot(z, Wc_ref[...],
                         preferred_element_type=jnp.float32) + bc_ref[...]


def _row_spec(width):
    return pl.BlockSpec((_RB, width), lambda i: (i, 0))


def _degp_spec():
    return pl.BlockSpec((NW, _RB), lambda i: (0, i))


def _aggp_spec():
    return pl.BlockSpec((NC, _RB, F), lambda i: (0, i, 0))


def _full(shape):
    return pl.BlockSpec(shape, lambda i: tuple(0 for _ in shape))


def kernel(g, x, W_est, b_est, W1, b1, W2, b2, Wc, bc):
    src = jnp.asarray(g[0], jnp.int32)
    dst = jnp.asarray(g[1], jnp.int32)
    zeros_deg = jnp.zeros((NP,), jnp.float32)
    zeros_f = jnp.zeros((NP, F), jnp.float32)

    degO_p, degI_p = _deg_kernel(src, dst, zeros_deg)
    degO_p = degO_p.reshape(NW, NP)
    degI_p = degI_p.reshape(NW, NP)

    xa = pl.pallas_call(
        _scale_x_body,
        grid=(_GRID,),
        in_specs=[_row_spec(F), _degp_spec()],
        out_specs=_row_spec(F),
        out_shape=jax.ShapeDtypeStruct((NP, F), jnp.float32),
    )(x, degO_p)

    agg0_p = _agg_kernel(src, dst, xa, zeros_f)

    s, h1a = pl.pallas_call(
        _mid_body,
        grid=(_GRID,),
        in_specs=[
            _aggp_spec(), _degp_spec(), _degp_spec(),
            _full((F, 1)), _full((1, 1)), _full((F, F)), _full((1, F)),
        ],
        out_specs=[_row_spec(1), _row_spec(F)],
        out_shape=[
            jax.ShapeDtypeStruct((N, 1), jnp.float32),
            jax.ShapeDtypeStruct((NP, F), jnp.float32),
        ],
    )(agg0_p, degO_p, degI_p, W_est, b_est.reshape(1, 1), W1, b1.reshape(1, F))

    agg1_p = _agg_kernel(src, dst, h1a, zeros_f)

    z, y = pl.pallas_call(
        _final_body,
        grid=(_GRID,),
        in_specs=[
            _aggp_spec(), _degp_spec(),
            _full((F, F)), _full((1, F)), _full((F, 1)), _full((1, 1)),
        ],
        out_specs=[_row_spec(F), _row_spec(1)],
        out_shape=[
            jax.ShapeDtypeStruct((N, F), jnp.float32),
            jax.ShapeDtypeStruct((N, 1), jnp.float32),
        ],
    )(agg1_p, degI_p, W2, b2.reshape(1, F), Wc, bc.reshape(1, 1))

    return (s, z, y)
